# baseline (speedup 1.0000x reference)
import dataclasses
import functools

import jax
import jax.numpy as jnp
from jax import lax
from jax.experimental import pallas as pl
from jax.experimental.pallas import tpu as pltpu
from jax.experimental.pallas import tpu_sc as plsc

NC = 2
NS = 16
NW = NC * NS
LANES = 16
CHUNK = 128
NBUF = 2
D = 128


def _round_up(v, m):
    return (v + m - 1) // m * m


def _sc_aggregate(table, src, dst, n_out, v_pad):
    vps = v_pad // NS
    e_pad = src.shape[0]
    per_w = e_pad // NW
    chunks = per_w // CHUNK
    n_pad = _round_up(n_out + 1, NS * 8)
    rpw = n_pad // NS

    mesh = plsc.VectorSubcoreMesh(core_axis_name="c", subcore_axis_name="s")
    cp = pltpu.CompilerParams()
    if "needs_layout_passes" in pltpu.CompilerParams.__dataclass_fields__:
        cp = dataclasses.replace(cp, needs_layout_passes=False)

    @functools.partial(
        pl.kernel,
        mesh=mesh,
        compiler_params=cp,
        out_type=(
            jax.ShapeDtypeStruct((NC, n_pad, D), jnp.float32),
            jax.ShapeDtypeStruct((NC * n_pad,), jnp.float32),
        ),
        scratch_types=[
            [pltpu.VMEM((CHUNK,), jnp.int32)] * NBUF,
            [pltpu.VMEM((CHUNK,), jnp.int32)] * NBUF,
            [pltpu.VMEM((CHUNK, D), jnp.float32)] * NBUF,
            pltpu.VMEM((n_pad,), jnp.float32),
            pltpu.VMEM((NS * (n_pad // NS),), jnp.float32),
            pltpu.VMEM_SHARED((n_pad, D), jnp.float32),
            pltpu.VMEM_SHARED((NS * n_pad,), jnp.float32),
            pltpu.VMEM_SHARED((v_pad, D), jnp.float32),
            [pltpu.SemaphoreType.DMA] * NBUF,
            [pltpu.SemaphoreType.DMA] * NBUF,
            [pltpu.SemaphoreType.DMA] * NBUF,
            pltpu.SemaphoreType.DMA,
        ],
    )
    def agg(table_hbm, src_hbm, dst_hbm, sum_out, cnt_out,
            srcb, dstb, rows, cnt_l, bounce, acc_s, cnt_stage, table_s,
            semi, semg, sems, semt):
        c = lax.axis_index("c")
        s = lax.axis_index("s")
        wid = s * NC + c
        ebase = wid * per_w
        zero16 = jnp.zeros((LANES,), jnp.float32)
        one16 = jnp.ones((LANES,), jnp.float32)

        def idx_pair(ch, b):
            off = ebase + ch * CHUNK
            return (
                pltpu.make_async_copy(src_hbm.at[pl.ds(off, CHUNK)], srcb[b], semi[b]),
                pltpu.make_async_copy(dst_hbm.at[pl.ds(off, CHUNK)], dstb[b], semi[b]),
            )

        def gather(ch, b):
            return pltpu.make_async_copy(table_s.at[srcb[b]], rows[b], semg[b])

        def gather_hbm(ch, b):
            return pltpu.make_async_copy(table_hbm.at[srcb[b]], rows[b], semg[b])

        def scatter(ch, b):
            return pltpu.make_async_copy(rows[b], acc_s.at[dstb[b]], sems[b])

        for b in range(NBUF):
            for cp in idx_pair(b, b):
                cp.start()
        @pl.when(c != 0)
        def _():
            pltpu.async_copy(
                table_hbm.at[pl.ds(s * vps, vps)],
                table_s.at[pl.ds(s * vps, vps)], semt)

        @pl.loop(0, CHUNK)
        def _(r):
            @pl.loop(0, D // LANES)
            def _(cc):
                rows[0][r, pl.ds(cc * LANES, LANES)] = zero16

        @pl.loop(0, n_pad, step=LANES)
        def _(r):
            cnt_l[pl.ds(r, LANES)] = zero16

        base = s * rpw
        off = 0
        while off < rpw:
            m = min(CHUNK, rpw - off)
            pltpu.sync_copy(rows[0].at[pl.ds(0, m)], acc_s.at[pl.ds(base + off, m)])
            off += m

        @pl.when(c != 0)
        def _():
            pltpu.make_async_copy(
                table_hbm.at[pl.ds(s * vps, vps)],
                table_s.at[pl.ds(s * vps, vps)], semt).wait()

        plsc.subcore_barrier()

        @pl.loop(0, chunks, step=NBUF)
        def _(i):
            for b in range(NBUF):
                for cp in idx_pair(i + b, b):
                    cp.wait()

                @pl.when(c == 0)
                def _(b=b):
                    gather_hbm(i + b, b).start()

                @pl.when(c != 0)
                def _(b=b):
                    gather(i + b, b).start()
            for b in range(NBUF):
                gather(i + b, b).wait()
                scatter(i + b, b).start(add=True)
                for j in range(CHUNK // LANES):
                    d16 = dstb[b][pl.ds(j * LANES, LANES)]
                    plsc.addupdate_scatter(cnt_l, [d16], one16)
            for b in range(NBUF):
                ch = i + b
                scatter(ch, b).wait()

                @pl.when(ch + NBUF < chunks)
                def _():
                    for cp in idx_pair(ch + NBUF, b):
                        cp.start()

        pltpu.sync_copy(cnt_l, cnt_stage.at[pl.ds(s * n_pad, n_pad)])
        plsc.subcore_barrier()

        pltpu.sync_copy(acc_s.at[pl.ds(base, rpw)], sum_out.at[c, pl.ds(base, rpw)])
        for k in range(NS):
            pltpu.sync_copy(cnt_stage.at[pl.ds(k * n_pad + base, rpw)],
                            bounce.at[pl.ds(k * rpw, rpw)])

        @pl.loop(0, rpw, step=LANES)
        def _(j):
            v = bounce[pl.ds(j, LANES)]
            for k in range(1, NS):
                v = v + bounce[pl.ds(k * rpw + j, LANES)]
            bounce[pl.ds(j, LANES)] = v

        pltpu.sync_copy(bounce.at[pl.ds(0, rpw)],
                        cnt_out.at[pl.ds(c * n_pad + base, rpw)])

    return agg(table, src, dst)


def _dense_body(sp_ref, cp_ref, xt_ref, wl_ref, bl_ref, wr_ref, o_ref):
    ssum = sp_ref[0] + sp_ref[1]
    cnt = jnp.sum(cp_ref[...], axis=1, keepdims=True)
    mean = ssum / jnp.maximum(cnt, 1.0)
    h = jnp.dot(mean, wl_ref[...], preferred_element_type=jnp.float32)
    h = h + bl_ref[...]
    h = h + jnp.dot(xt_ref[...], wr_ref[...], preferred_element_type=jnp.float32)
    o_ref[...] = jnp.maximum(h, 0.0)


def _dense1(sum_parts, cnt_parts, x_full, m, WlT, bl, WrT):
    bm = 1000
    return pl.pallas_call(
        _dense_body,
        grid=(m // bm,),
        in_specs=[
            pl.BlockSpec((NC, bm, D), lambda i: (0, i, 0)),
            pl.BlockSpec((bm, NC), lambda i: (i, 0)),
            pl.BlockSpec((bm, D), lambda i: (i, 0)),
            pl.BlockSpec((D, D), lambda i: (0, 0)),
            pl.BlockSpec((1, D), lambda i: (0, 0)),
            pl.BlockSpec((D, D), lambda i: (0, 0)),
        ],
        out_specs=pl.BlockSpec((bm, D), lambda i: (i, 0)),
        out_shape=jax.ShapeDtypeStruct((m, D), jnp.float32),
    )(sum_parts, cnt_parts, x_full, WlT, bl.reshape(1, D), WrT)


def _dense2_body(sp_ref, cp_ref, xt_ref, wl_ref, bl_ref, wr_ref, o_ref, ls_ref):
    ssum = sp_ref[0] + sp_ref[1]
    cnt = jnp.sum(cp_ref[...], axis=1, keepdims=True)
    mean = ssum / jnp.maximum(cnt, 1.0)
    o = jnp.dot(mean, wl_ref[...], preferred_element_type=jnp.float32)
    o = o + bl_ref[...]
    o = o + jnp.dot(xt_ref[...], wr_ref[...], preferred_element_type=jnp.float32)
    o_ref[...] = o
    mx = jnp.max(o, axis=-1, keepdims=True)
    e = jnp.exp(o - mx)
    lse = jnp.log(jnp.sum(e, axis=-1, keepdims=True)) + mx
    ls_ref[...] = o - lse


def _dense2(sum_parts, cnt_parts, h_full, m, WlT, bl, WrT):
    return pl.pallas_call(
        _dense2_body,
        grid=(1,),
        in_specs=[
            pl.BlockSpec((NC, m, D), lambda i: (0, 0, 0)),
            pl.BlockSpec((m, NC), lambda i: (0, 0)),
            pl.BlockSpec((m, D), lambda i: (0, 0)),
            pl.BlockSpec((D, D), lambda i: (0, 0)),
            pl.BlockSpec((1, D), lambda i: (0, 0)),
            pl.BlockSpec((D, D), lambda i: (0, 0)),
        ],
        out_specs=(
            pl.BlockSpec((m, D), lambda i: (0, 0)),
            pl.BlockSpec((m, D), lambda i: (0, 0)),
        ),
        out_shape=(
            jax.ShapeDtypeStruct((m, D), jnp.float32),
            jax.ShapeDtypeStruct((m, D), jnp.float32),
        ),
    )(sum_parts, cnt_parts, h_full, WlT, bl.reshape(1, D), WrT)


def _pad_edges(edge_index, n_out):
    src = edge_index[0].astype(jnp.int32)
    dst = edge_index[1].astype(jnp.int32)
    e = src.shape[0]
    e_pad = _round_up(e, NW * CHUNK * NBUF)
    pad = e_pad - e
    if pad:
        src = jnp.concatenate([src, jnp.zeros((pad,), jnp.int32)])
        dst = jnp.concatenate([dst, jnp.full((pad,), n_out, jnp.int32)])
    return src, dst


def kernel(x, edge_index_1, edge_index_2, Wl1, bl1, Wr1, Wl2, bl2, Wr2):
    n1, n2 = 5000, 1000
    src1, dst1 = _pad_edges(edge_index_1, n1)
    src2, dst2 = _pad_edges(edge_index_2, n2)

    s1, c1 = _sc_aggregate(x, src1, dst1, n1, _round_up(n1, NS * 8))
    h = _dense1(s1, c1.reshape(NC, -1).T, x, n1, Wl1.T, bl1, Wr1.T)

    s2, c2 = _sc_aggregate(h, src2, dst2, n2, _round_up(n2, NS * 8))
    out, ls = _dense2(s2, c2.reshape(NC, -1).T, h, n2, Wl2.T, bl2, Wr2.T)
    return (out, ls)


# speedup vs baseline: 1.0365x; 1.0365x over previous
import dataclasses
import functools

import jax
import jax.numpy as jnp
from jax import lax
from jax.experimental import pallas as pl
from jax.experimental.pallas import tpu as pltpu
from jax.experimental.pallas import tpu_sc as plsc

NC = 2
NS = 16
NW = NC * NS
LANES = 16
CHUNK = 128
NBUF = 2
D = 128


def _round_up(v, m):
    return (v + m - 1) // m * m


def _sc_aggregate(table, edges, n_out, v_pad):
    vps = v_pad // NS
    e_total = edges.shape[0] // 2
    per_w = e_total // NW
    assert per_w * NW == e_total and per_w % 8 == 0
    chunks_full = per_w // CHUNK
    tail = per_w - chunks_full * CHUNK
    assert tail % 8 == 0
    main = (chunks_full // NBUF) * NBUF
    rem = chunks_full - main
    tailb = max(tail, 8)
    n_pad = _round_up(n_out + 1, NS * 8)
    rpw = n_pad // NS

    mesh = plsc.VectorSubcoreMesh(core_axis_name="c", subcore_axis_name="s")
    cp = pltpu.CompilerParams()
    if "needs_layout_passes" in pltpu.CompilerParams.__dataclass_fields__:
        cp = dataclasses.replace(cp, needs_layout_passes=False)

    @functools.partial(
        pl.kernel,
        mesh=mesh,
        compiler_params=cp,
        out_type=(
            jax.ShapeDtypeStruct((NC, n_pad, D), jnp.float32),
            jax.ShapeDtypeStruct((NC * n_pad,), jnp.float32),
        ),
        scratch_types=[
            [pltpu.VMEM((CHUNK,), jnp.int32)] * NBUF,
            [pltpu.VMEM((CHUNK,), jnp.int32)] * NBUF,
            pltpu.VMEM((tailb,), jnp.int32),
            pltpu.VMEM((tailb,), jnp.int32),
            [pltpu.VMEM((CHUNK, D), jnp.float32)] * NBUF,
            pltpu.VMEM((n_pad,), jnp.float32),
            pltpu.VMEM((NS * (n_pad // NS),), jnp.float32),
            pltpu.VMEM_SHARED((n_pad, D), jnp.float32),
            pltpu.VMEM_SHARED((NS * n_pad,), jnp.float32),
            pltpu.VMEM_SHARED((v_pad, D), jnp.float32),
            [pltpu.SemaphoreType.DMA] * NBUF,
            [pltpu.SemaphoreType.DMA] * NBUF,
            [pltpu.SemaphoreType.DMA] * NBUF,
            pltpu.SemaphoreType.DMA,
        ],
    )
    def agg(table_hbm, edges_hbm, sum_out, cnt_out,
            srcb, dstb, srct, dstt, rows, cnt_l, bounce, acc_s, cnt_stage,
            table_s, semi, semg, sems, semt):
        c = lax.axis_index("c")
        s = lax.axis_index("s")
        wid = s * NC + c
        ebase = wid * per_w
        zero16 = jnp.zeros((LANES,), jnp.float32)
        one16 = jnp.ones((LANES,), jnp.float32)

        def idx_pair(ch, b):
            off = ebase + ch * CHUNK
            return (
                pltpu.make_async_copy(
                    edges_hbm.at[pl.ds(off, CHUNK)], srcb[b], semi[b]),
                pltpu.make_async_copy(
                    edges_hbm.at[pl.ds(e_total + off, CHUNK)], dstb[b], semi[b]),
            )

        def gather(ch, b):
            return pltpu.make_async_copy(table_s.at[srcb[b]], rows[b], semg[b])

        def scatter(ch, b):
            return pltpu.make_async_copy(rows[b], acc_s.at[dstb[b]], sems[b])

        if main > 0:
            for b in range(NBUF):
                for cp in idx_pair(b, b):
                    cp.start()
        tcp = pltpu.async_copy(
            table_hbm.at[pl.ds(s * vps, vps)], table_s.at[pl.ds(s * vps, vps)],
            semt)

        @pl.loop(0, CHUNK)
        def _(r):
            @pl.loop(0, D // LANES)
            def _(cc):
                rows[0][r, pl.ds(cc * LANES, LANES)] = zero16

        @pl.loop(0, n_pad, step=LANES)
        def _(r):
            cnt_l[pl.ds(r, LANES)] = zero16

        base = s * rpw
        off = 0
        while off < rpw:
            m = min(CHUNK, rpw - off)
            pltpu.sync_copy(rows[0].at[pl.ds(0, m)], acc_s.at[pl.ds(base + off, m)])
            off += m

        tcp.wait()
        plsc.subcore_barrier()

        if main > 0:
            @pl.loop(0, main, step=NBUF)
            def _(i):
                for b in range(NBUF):
                    for cp in idx_pair(i + b, b):
                        cp.wait()
                    gather(i + b, b).start()
                for b in range(NBUF):
                    gather(i + b, b).wait()
                    scatter(i + b, b).start(add=True)
                    for j in range(CHUNK // LANES):
                        d16 = dstb[b][pl.ds(j * LANES, LANES)]
                        plsc.addupdate_scatter(cnt_l, [d16], one16)
                for b in range(NBUF):
                    ch = i + b
                    scatter(ch, b).wait()

                    @pl.when(ch + NBUF < main)
                    def _():
                        for cp in idx_pair(ch + NBUF, b):
                            cp.start()

        for k in range(rem):
            off = ebase + (main + k) * CHUNK
            pltpu.sync_copy(edges_hbm.at[pl.ds(off, CHUNK)], srcb[0])
            pltpu.sync_copy(edges_hbm.at[pl.ds(e_total + off, CHUNK)], dstb[0])
            pltpu.async_copy(table_s.at[srcb[0]], rows[0], semg[0]).wait()
            pltpu.sync_copy(rows[0], acc_s.at[dstb[0]], add=True)
            for j in range(CHUNK // LANES):
                d16 = dstb[0][pl.ds(j * LANES, LANES)]
                plsc.addupdate_scatter(cnt_l, [d16], one16)
        if tail:
            off = ebase + chunks_full * CHUNK
            pltpu.sync_copy(edges_hbm.at[pl.ds(off, tail)], srct)
            pltpu.sync_copy(edges_hbm.at[pl.ds(e_total + off, tail)], dstt)
            pltpu.async_copy(
                table_s.at[srct], rows[0].at[pl.ds(0, tail)], semg[0]).wait()
            pltpu.sync_copy(rows[0].at[pl.ds(0, tail)], acc_s.at[dstt], add=True)
            for j in range(tail // LANES):
                d16 = dstt[pl.ds(j * LANES, LANES)]
                plsc.addupdate_scatter(cnt_l, [d16], one16)

        pltpu.sync_copy(cnt_l, cnt_stage.at[pl.ds(s * n_pad, n_pad)])
        plsc.subcore_barrier()

        pltpu.sync_copy(acc_s.at[pl.ds(base, rpw)], sum_out.at[c, pl.ds(base, rpw)])
        for k in range(NS):
            pltpu.sync_copy(cnt_stage.at[pl.ds(k * n_pad + base, rpw)],
                            bounce.at[pl.ds(k * rpw, rpw)])

        @pl.loop(0, rpw, step=LANES)
        def _(j):
            v = bounce[pl.ds(j, LANES)]
            for k in range(1, NS):
                v = v + bounce[pl.ds(k * rpw + j, LANES)]
            bounce[pl.ds(j, LANES)] = v

        pltpu.sync_copy(bounce.at[pl.ds(0, rpw)],
                        cnt_out.at[pl.ds(c * n_pad + base, rpw)])

    return agg(table, edges)


def _dense_body(sp_ref, cp_ref, xt_ref, wl_ref, bl_ref, wr_ref, o_ref):
    ssum = sp_ref[0] + sp_ref[1]
    cnt = jnp.sum(cp_ref[...], axis=1, keepdims=True)
    mean = ssum / jnp.maximum(cnt, 1.0)
    h = jnp.dot(mean, wl_ref[...], preferred_element_type=jnp.float32)
    h = h + bl_ref[...]
    h = h + jnp.dot(xt_ref[...], wr_ref[...], preferred_element_type=jnp.float32)
    o_ref[...] = jnp.maximum(h, 0.0)


def _dense1(sum_parts, cnt_parts, x_full, m, WlT, bl, WrT):
    bm = 1000
    return pl.pallas_call(
        _dense_body,
        grid=(m // bm,),
        in_specs=[
            pl.BlockSpec((NC, bm, D), lambda i: (0, i, 0)),
            pl.BlockSpec((bm, NC), lambda i: (i, 0)),
            pl.BlockSpec((bm, D), lambda i: (i, 0)),
            pl.BlockSpec((D, D), lambda i: (0, 0)),
            pl.BlockSpec((1, D), lambda i: (0, 0)),
            pl.BlockSpec((D, D), lambda i: (0, 0)),
        ],
        out_specs=pl.BlockSpec((bm, D), lambda i: (i, 0)),
        out_shape=jax.ShapeDtypeStruct((m, D), jnp.float32),
    )(sum_parts, cnt_parts, x_full, WlT, bl.reshape(1, D), WrT)


def _dense2_body(sp_ref, cp_ref, xt_ref, wl_ref, bl_ref, wr_ref, o_ref, ls_ref):
    ssum = sp_ref[0] + sp_ref[1]
    cnt = jnp.sum(cp_ref[...], axis=1, keepdims=True)
    mean = ssum / jnp.maximum(cnt, 1.0)
    o = jnp.dot(mean, wl_ref[...], preferred_element_type=jnp.float32)
    o = o + bl_ref[...]
    o = o + jnp.dot(xt_ref[...], wr_ref[...], preferred_element_type=jnp.float32)
    o_ref[...] = o
    mx = jnp.max(o, axis=-1, keepdims=True)
    e = jnp.exp(o - mx)
    lse = jnp.log(jnp.sum(e, axis=-1, keepdims=True)) + mx
    ls_ref[...] = o - lse


def _dense2(sum_parts, cnt_parts, h_full, m, WlT, bl, WrT):
    return pl.pallas_call(
        _dense2_body,
        grid=(1,),
        in_specs=[
            pl.BlockSpec((NC, m, D), lambda i: (0, 0, 0)),
            pl.BlockSpec((m, NC), lambda i: (0, 0)),
            pl.BlockSpec((m, D), lambda i: (0, 0)),
            pl.BlockSpec((D, D), lambda i: (0, 0)),
            pl.BlockSpec((1, D), lambda i: (0, 0)),
            pl.BlockSpec((D, D), lambda i: (0, 0)),
        ],
        out_specs=(
            pl.BlockSpec((m, D), lambda i: (0, 0)),
            pl.BlockSpec((m, D), lambda i: (0, 0)),
        ),
        out_shape=(
            jax.ShapeDtypeStruct((m, D), jnp.float32),
            jax.ShapeDtypeStruct((m, D), jnp.float32),
        ),
    )(sum_parts, cnt_parts, h_full, WlT, bl.reshape(1, D), WrT)


def kernel(x, edge_index_1, edge_index_2, Wl1, bl1, Wr1, Wl2, bl2, Wr2):
    n1, n2 = 5000, 1000
    e1 = edge_index_1.astype(jnp.int32).reshape(-1)
    e2 = edge_index_2.astype(jnp.int32).reshape(-1)

    s1, c1 = _sc_aggregate(x, e1, n1, _round_up(n1, NS * 8))
    h = _dense1(s1, c1.reshape(NC, -1).T, x, n1, Wl1.T, bl1, Wr1.T)

    s2, c2 = _sc_aggregate(h, e2, n2, _round_up(n2, NS * 8))
    out, ls = _dense2(s2, c2.reshape(NC, -1).T, h, n2, Wl2.T, bl2, Wr2.T)
    return (out, ls)
